# baseline (speedup 1.0000x reference)
import jax
import jax.numpy as jnp
from jax import lax
from jax.experimental import pallas as pl
from jax.experimental.pallas import tpu as pltpu
from jax.experimental.pallas import tpu_sc as plsc

B = 4
T = 2048
H = 512
N = 256
NSPAN = B * N
OUT_D = 4 * H

NC = 2
NS = 16
NW = NC * NS
SPW = NSPAN // NW
CH = 8
NCHUNK = SPW // CH
L = 16
G = H // L

_HALF_OFF = (0, 0, H, H)


def _body(x_hbm, i_hbm, j_hbm, out_hbm, iv, jv, idx_v, mult_v, rows_v,
          out_v, sem_g0, sem_g1, sem_w0, sem_w1):
  sem_g = (sem_g0, sem_g1)
  sem_w = (sem_w0, sem_w1)
  wid = lax.axis_index("s") * NC + lax.axis_index("c")
  base = wid * SPW
  row_base = (wid // (N // SPW)) * T

  pltpu.sync_copy(i_hbm.at[pl.ds(base, SPW)], iv)
  pltpu.sync_copy(j_hbm.at[pl.ds(base, SPW)], jv)

  def compute_pair(p):
    pb = p % 2
    i16 = iv[pl.ds(p * L, L)]
    j16 = jv[pl.ds(p * L, L)]
    one = jnp.full((L,), 1.0, jnp.float32)
    zero = jnp.zeros((L,), jnp.float32)
    k16 = jnp.where((i16 != 0) | (j16 != 0), one, zero)
    a16 = jnp.where(i16 >= 1, k16, zero)
    c16 = jnp.where(j16 < T - 1, k16, zero)
    idx_v[pl.ds(pb * 64 + 0 * L, L)] = j16 + row_base
    idx_v[pl.ds(pb * 64 + 1 * L, L)] = jnp.maximum(i16 - 1, 0) + row_base
    idx_v[pl.ds(pb * 64 + 2 * L, L)] = i16 + row_base
    idx_v[pl.ds(pb * 64 + 3 * L, L)] = jnp.minimum(j16 + 1, T - 1) + row_base
    mult_v[pl.ds(pb * 48 + 0 * L, L)] = k16
    mult_v[pl.ds(pb * 48 + 1 * L, L)] = a16
    mult_v[pl.ds(pb * 48 + 2 * L, L)] = c16

  def fire(c):
    pb = (c // 2) % 2
    rb = c % 2
    off = (c % 2) * CH
    cps = []
    for t in range(4):
      src = x_hbm.at[idx_v.at[pl.ds(pb * 64 + t * L + off, CH)],
                     pl.ds(_HALF_OFF[t], H)]
      dst = rows_v.at[rb, pl.ds(t * CH, CH)]
      cps.append(pltpu.async_copy(src, dst, sem_g[rb]))
    return cps

  def compute(c):
    pb = (c // 2) % 2
    rb = c % 2
    lane0 = (c % 2) * CH

    def span_body(s, _):
      lane = pb * 48 + lane0 + s
      kk = plsc.load_gather(mult_v, [jnp.full((L,), 0, jnp.int32) + lane])
      aa = plsc.load_gather(mult_v, [jnp.full((L,), L, jnp.int32) + lane])
      cc = plsc.load_gather(mult_v, [jnp.full((L,), 2 * L, jnp.int32) + lane])

      def grp_body(g, _):
        off = g * L
        fend = rows_v[rb, s, pl.ds(off, L)]
        fpre = rows_v[rb, CH + s, pl.ds(off, L)]
        bsta = rows_v[rb, 2 * CH + s, pl.ds(off, L)]
        bpos = rows_v[rb, 3 * CH + s, pl.ds(off, L)]
        fpa = fpre * aa
        bpc = bpos * cc
        out_v[rb, s, pl.ds(off, L)] = fend * kk - fpa
        out_v[rb, s, pl.ds(H + off, L)] = bsta * kk - bpc
        out_v[rb, s, pl.ds(2 * H + off, L)] = fpa
        out_v[rb, s, pl.ds(3 * H + off, L)] = bpc
        return 0

      lax.fori_loop(0, G, grp_body, 0, unroll=4)
      return 0

    lax.fori_loop(0, CH, span_body, 0)

  def write(c):
    rb = c % 2
    return pltpu.async_copy(out_v.at[rb],
                            out_hbm.at[pl.ds(base + c * CH, CH)], sem_w[rb])

  compute_pair(0)
  pend_g = {0: fire(0)}
  pend_w = {}
  for c in range(NCHUNK):
    if c + 1 < NCHUNK:
      if (c + 1) % 2 == 0:
        compute_pair((c + 1) // 2)
      pend_g[c + 1] = fire(c + 1)
    if c >= 2:
      pend_w[c - 2].wait()
    for cp in pend_g[c]:
      cp.wait()
    compute(c)
    pend_w[c] = write(c)
  pend_w[NCHUNK - 2].wait()
  pend_w[NCHUNK - 1].wait()


@jax.jit
def _launch(x2, i_flat, j_flat):
  mesh = plsc.VectorSubcoreMesh(core_axis_name="c", subcore_axis_name="s")
  return pl.kernel(
      _body,
      out_type=jax.ShapeDtypeStruct((NSPAN, OUT_D), jnp.float32),
      mesh=mesh,
      compiler_params=pltpu.CompilerParams(needs_layout_passes=False),
      scratch_types=[
          pltpu.VMEM((SPW,), jnp.int32),
          pltpu.VMEM((SPW,), jnp.int32),
          pltpu.VMEM((128,), jnp.int32),
          pltpu.VMEM((96,), jnp.float32),
          pltpu.VMEM((2, 4 * CH, H), jnp.float32),
          pltpu.VMEM((2, CH, OUT_D), jnp.float32),
          pltpu.SemaphoreType.DMA,
          pltpu.SemaphoreType.DMA,
          pltpu.SemaphoreType.DMA,
          pltpu.SemaphoreType.DMA,
      ],
  )(x2, i_flat, j_flat)


def kernel(input, span_idxs):
  x2 = input.reshape(B * T, 2 * H)
  ij = span_idxs.reshape(NSPAN, 2)
  i_flat = ij[:, 0].astype(jnp.int32)
  j_flat = ij[:, 1].astype(jnp.int32)
  out = _launch(x2, i_flat, j_flat)
  return out.reshape(B, N, OUT_D)


# speedup vs baseline: 1.3428x; 1.3428x over previous
import jax
import jax.numpy as jnp
from jax import lax
from jax.experimental import pallas as pl
from jax.experimental.pallas import tpu as pltpu
from jax.experimental.pallas import tpu_sc as plsc

B = 4
T = 2048
H = 512
N = 256
NSPAN = B * N
OUT_D = 4 * H

NC = 2
NS = 16
NW = NC * NS
SPW = NSPAN // NW
CH = 16
NCHUNK = SPW // CH
L = 16
G = H // L


def _body(x_hbm, i_hbm, j_hbm, out_hbm, iv, jv, idx_v, rows_v, out_v,
          sem_g0, sem_g1, sem_w0, sem_w1):
  sem_g = (sem_g0, sem_g1)
  sem_w = (sem_w0, sem_w1)
  wid = lax.axis_index("s") * NC + lax.axis_index("c")
  base = wid * SPW
  row_base = (wid // (N // SPW)) * T

  pltpu.sync_copy(i_hbm.at[pl.ds(base, SPW)], iv)
  pltpu.sync_copy(j_hbm.at[pl.ds(base, SPW)], jv)

  def prep(c):
    i16 = iv[pl.ds(c * CH, L)]
    j16 = jv[pl.ds(c * CH, L)]
    idx_v[pl.ds(c * 64 + 0 * L, L)] = j16 + row_base
    idx_v[pl.ds(c * 64 + 1 * L, L)] = i16 + row_base
    idx_v[pl.ds(c * 64 + 2 * L, L)] = jnp.maximum(i16 - 1, 0) + row_base
    idx_v[pl.ds(c * 64 + 3 * L, L)] = jnp.minimum(j16 + 1, T - 1) + row_base

  def fire(c):
    rb = c % 2
    return [
        pltpu.async_copy(
            x_hbm.at[idx_v.at[pl.ds(c * 64 + 0 * L, CH)], pl.ds(0, H)],
            rows_v.at[rb, pl.ds(0, CH)], sem_g[rb]),
        pltpu.async_copy(
            x_hbm.at[idx_v.at[pl.ds(c * 64 + 1 * L, CH)], pl.ds(H, H)],
            rows_v.at[rb, pl.ds(CH, CH)], sem_g[rb]),
        pltpu.async_copy(
            x_hbm.at[idx_v.at[pl.ds(c * 64 + 2 * L, CH)], pl.ds(0, H)],
            out_v.at[rb, :, pl.ds(2 * H, H)], sem_g[rb]),
        pltpu.async_copy(
            x_hbm.at[idx_v.at[pl.ds(c * 64 + 3 * L, CH)], pl.ds(H, H)],
            out_v.at[rb, :, pl.ds(3 * H, H)], sem_g[rb]),
    ]

  def fixup(c):
    rb = c % 2
    i16 = iv[pl.ds(c * CH, L)]
    j16 = jv[pl.ds(c * CH, L)]
    need = jnp.where((i16 == 0) | (j16 >= T - 1),
                     jnp.full((L,), 1, jnp.int32), jnp.zeros((L,), jnp.int32))
    any_need = lax.reduce_max(need, (0,))

    @pl.when(any_need > 0)
    def _():
      one = jnp.full((L,), 1.0, jnp.float32)
      zero = jnp.zeros((L,), jnp.float32)
      k16 = jnp.where((i16 != 0) | (j16 != 0), one, zero)
      a16 = jnp.where(i16 >= 1, k16, zero)
      c16 = jnp.where(j16 < T - 1, k16, zero)

      def span_fix(s, _):
        sidx = jnp.full((L,), s, jnp.int32)
        kk = k16.at[sidx].get(mode="promise_in_bounds")
        aa = a16.at[sidx].get(mode="promise_in_bounds")
        cc = c16.at[sidx].get(mode="promise_in_bounds")

        def fx(g, _):
          off = g * L
          rows_v[rb, s, pl.ds(off, L)] = rows_v[rb, s, pl.ds(off, L)] * kk
          rows_v[rb, CH + s, pl.ds(off, L)] = (
              rows_v[rb, CH + s, pl.ds(off, L)] * kk)
          out_v[rb, s, pl.ds(2 * H + off, L)] = (
              out_v[rb, s, pl.ds(2 * H + off, L)] * aa)
          out_v[rb, s, pl.ds(3 * H + off, L)] = (
              out_v[rb, s, pl.ds(3 * H + off, L)] * cc)
          return 0

        lax.fori_loop(0, G, fx, 0, unroll=2)
        return 0

      lax.fori_loop(0, CH, span_fix, 0)

  def compute(c):
    rb = c % 2

    def span_body(s, _):
      def grp_body(g, _):
        off = g * L
        fend = rows_v[rb, s, pl.ds(off, L)]
        bsta = rows_v[rb, CH + s, pl.ds(off, L)]
        fpre = out_v[rb, s, pl.ds(2 * H + off, L)]
        bpos = out_v[rb, s, pl.ds(3 * H + off, L)]
        out_v[rb, s, pl.ds(off, L)] = fend - fpre
        out_v[rb, s, pl.ds(H + off, L)] = bsta - bpos
        return 0

      lax.fori_loop(0, G, grp_body, 0, unroll=4)
      return 0

    lax.fori_loop(0, CH, span_body, 0)

  def write(c):
    rb = c % 2
    return pltpu.async_copy(out_v.at[rb],
                            out_hbm.at[pl.ds(base + c * CH, CH)], sem_w[rb])

  prep(0)
  pend0 = fire(0)
  prep(1)
  pend1 = fire(1)
  for cp in pend0:
    cp.wait()
  fixup(0)
  compute(0)
  w0 = write(0)
  for cp in pend1:
    cp.wait()
  fixup(1)
  compute(1)
  w1 = write(1)
  w0.wait()
  w1.wait()


@jax.jit
def _launch(x2, i_flat, j_flat):
  mesh = plsc.VectorSubcoreMesh(core_axis_name="c", subcore_axis_name="s")
  return pl.kernel(
      _body,
      out_type=jax.ShapeDtypeStruct((NSPAN, OUT_D), jnp.float32),
      mesh=mesh,
      compiler_params=pltpu.CompilerParams(needs_layout_passes=False),
      scratch_types=[
          pltpu.VMEM((SPW,), jnp.int32),
          pltpu.VMEM((SPW,), jnp.int32),
          pltpu.VMEM((128,), jnp.int32),
          pltpu.VMEM((2, 2 * CH, H), jnp.float32),
          pltpu.VMEM((2, CH, OUT_D), jnp.float32),
          pltpu.SemaphoreType.DMA,
          pltpu.SemaphoreType.DMA,
          pltpu.SemaphoreType.DMA,
          pltpu.SemaphoreType.DMA,
      ],
  )(x2, i_flat, j_flat)


def kernel(input, span_idxs):
  x2 = input.reshape(B * T, 2 * H)
  ij = span_idxs.reshape(NSPAN, 2)
  i_flat = ij[:, 0].astype(jnp.int32)
  j_flat = ij[:, 1].astype(jnp.int32)
  out = _launch(x2, i_flat, j_flat)
  return out.reshape(B, N, OUT_D)
